# baseline (speedup 1.0000x reference)
import functools

import jax
import jax.numpy as jnp
from jax.experimental import pallas as pl
from jax.experimental.pallas import tpu as pltpu

_EPS = 1e-5


def _block_kernel(x_ref, wc_ref, w1t_ref, g1_ref, b1_ref, g2_ref, b2_ref,
                  o_ref, y1_ref, acc1_ref, acc2_ref, sc1_ref, sh1_ref,
                  sc2_ref, sh2_ref, *, B, H, W, Cout, count):
    HW = H * W
    LB = B * HW
    ph = pl.program_id(0)
    i = pl.program_id(1)

    @pl.when(jnp.logical_and(ph == 0, i == 0))
    def _init():
        acc1_ref[...] = jnp.zeros_like(acc1_ref)
        acc2_ref[...] = jnp.zeros_like(acc2_ref)

    @pl.when(ph == 0)
    def _conv3():
        xb = jnp.concatenate([x_ref[b] for b in range(B)],
                             axis=1).astype(jnp.bfloat16)
        lane = jax.lax.broadcasted_iota(jnp.int32, (1, LB), 1)
        wpos = lane % W
        hpos = (lane // W) % H
        left = jnp.where(
            wpos >= 1,
            jnp.concatenate([xb[:, LB - 1:], xb[:, :LB - 1]], axis=1),
            jnp.bfloat16(0))
        right = jnp.where(
            wpos <= W - 2,
            jnp.concatenate([xb[:, 1:], xb[:, :1]], axis=1),
            jnp.bfloat16(0))
        Cin = xb.shape[0]
        dims = (((1,), (0,)), ((), ()))
        c = (jax.lax.dot_general(wc_ref[:, :Cin], left, dims,
                                 preferred_element_type=jnp.float32)
             + jax.lax.dot_general(wc_ref[:, Cin:2 * Cin], xb, dims,
                                   preferred_element_type=jnp.float32)
             + jax.lax.dot_general(wc_ref[:, 2 * Cin:], right, dims,
                                   preferred_element_type=jnp.float32))
        c0 = c[:Cout]
        c2 = c[2 * Cout:]
        up = jnp.where(
            hpos >= 1,
            jnp.concatenate([c0[:, LB - W:], c0[:, :LB - W]], axis=1), 0.0)
        dn = jnp.where(
            hpos <= H - 2,
            jnp.concatenate([c2[:, W:], c2[:, :W]], axis=1), 0.0)
        y = c[Cout:2 * Cout] + up + dn
        s = jnp.sum(y, axis=1, keepdims=True)
        ss = jnp.sum(y * y, axis=1, keepdims=True)
        acc1_ref[...] += jnp.concatenate([s, ss], axis=1)
        y1_ref[i] = y.astype(jnp.bfloat16)

    @pl.when(jnp.logical_and(ph == 1, i == 0))
    def _fold1():
        tot = acc1_ref[...]
        mean = tot[:, 0:1] / count
        var = tot[:, 1:2] / count - mean * mean
        sc = g1_ref[...] * jax.lax.rsqrt(var + _EPS)
        sc1_ref[...] = sc
        sh1_ref[...] = b1_ref[...] - mean * sc

    @pl.when(ph == 1)
    def _stats2():
        a = jnp.maximum(
            y1_ref[i].astype(jnp.float32) * sc1_ref[...] + sh1_ref[...], 0.0)
        z = jax.lax.dot_general(w1t_ref[...], a.astype(jnp.bfloat16),
                                (((1,), (0,)), ((), ())),
                                preferred_element_type=jnp.float32)
        s = jnp.sum(z, axis=1, keepdims=True)
        ss = jnp.sum(z * z, axis=1, keepdims=True)
        acc2_ref[...] += jnp.concatenate([s, ss], axis=1)
        y1_ref[i] = z.astype(jnp.bfloat16)

    @pl.when(jnp.logical_and(ph == 2, i == 0))
    def _fold2():
        tot = acc2_ref[...]
        mean = tot[:, 0:1] / count
        var = tot[:, 1:2] / count - mean * mean
        sc = g2_ref[...] * jax.lax.rsqrt(var + _EPS)
        sc2_ref[...] = sc
        sh2_ref[...] = b2_ref[...] - mean * sc

    @pl.when(ph == 2)
    def _emit():
        z = y1_ref[i].astype(jnp.float32)
        o = jnp.maximum(z * sc2_ref[...] + sh2_ref[...],
                        0.0).astype(jnp.bfloat16)
        for b in range(B):
            o_ref[b] = o[:, b * HW:(b + 1) * HW]


@jax.jit
def _forward(x_nchw, w3_hwio, w1, gamma1, beta1, gamma2, beta2):
    N, Cin, H, W = x_nchw.shape
    Cout = w3_hwio.shape[-1]
    HW = H * W
    B = 64 if N % 64 == 0 else (8 if N % 8 == 0 else 1)
    S = N // B
    LB = B * HW

    x3 = x_nchw.reshape(N, Cin, HW)
    wc = jnp.transpose(w3_hwio, (0, 3, 1, 2)).reshape(
        3 * Cout, 3 * Cin).astype(jnp.bfloat16)
    w1t = jnp.transpose(w1).astype(jnp.bfloat16)

    out3 = pl.pallas_call(
        functools.partial(_block_kernel, B=B, H=H, W=W, Cout=Cout,
                          count=N * HW),
        grid=(3, S),
        in_specs=[
            pl.BlockSpec((B, Cin, HW),
                         lambda p, i: (jnp.where(p == 0, i, 0), 0, 0)),
            pl.BlockSpec((3 * Cout, 3 * Cin), lambda p, i: (0, 0)),
            pl.BlockSpec((Cout, Cout), lambda p, i: (0, 0)),
            pl.BlockSpec((Cout, 1), lambda p, i: (0, 0)),
            pl.BlockSpec((Cout, 1), lambda p, i: (0, 0)),
            pl.BlockSpec((Cout, 1), lambda p, i: (0, 0)),
            pl.BlockSpec((Cout, 1), lambda p, i: (0, 0)),
        ],
        out_specs=pl.BlockSpec((B, Cout, HW),
                               lambda p, i: (jnp.where(p == 2, i, 0), 0, 0)),
        out_shape=jax.ShapeDtypeStruct((N, Cout, HW), jnp.bfloat16),
        scratch_shapes=[
            pltpu.VMEM((S, Cout, LB), jnp.bfloat16),
            pltpu.VMEM((Cout, 2), jnp.float32),
            pltpu.VMEM((Cout, 2), jnp.float32),
            pltpu.VMEM((Cout, 1), jnp.float32),
            pltpu.VMEM((Cout, 1), jnp.float32),
            pltpu.VMEM((Cout, 1), jnp.float32),
            pltpu.VMEM((Cout, 1), jnp.float32),
        ],
        compiler_params=pltpu.CompilerParams(
            dimension_semantics=("arbitrary", "arbitrary")),
    )(x3, wc, w1t, gamma1.reshape(-1, 1), beta1.reshape(-1, 1),
      gamma2.reshape(-1, 1), beta2.reshape(-1, 1))

    return out3.reshape(N, Cout, H, W).astype(jnp.float32)


def kernel(x_nchw, w3_hwio, w1, gamma1, beta1, gamma2, beta2):
    return _forward(x_nchw, w3_hwio, w1, gamma1, beta1, gamma2, beta2)


# speedup vs baseline: 1.1133x; 1.1133x over previous
import functools

import jax
import jax.numpy as jnp
from jax.experimental import pallas as pl
from jax.experimental.pallas import tpu as pltpu

_EPS = 1e-5


def _block_kernel(x_ref, wc_ref, w1t_ref, g1_ref, b1_ref, g2_ref, b2_ref,
                  o_ref, y1_ref, acc1_ref, acc2_ref, sc1_ref, sh1_ref,
                  sc2_ref, sh2_ref, *, B, H, W, Cout, count):
    HW = H * W
    LB = B * HW
    ph = pl.program_id(0)
    i = pl.program_id(1)

    @pl.when(jnp.logical_and(ph == 0, i == 0))
    def _init():
        acc1_ref[...] = jnp.zeros_like(acc1_ref)
        acc2_ref[...] = jnp.zeros_like(acc2_ref)

    @pl.when(ph == 0)
    def _conv3():
        xb = jnp.concatenate([x_ref[b] for b in range(B)],
                             axis=1).astype(jnp.bfloat16)
        lane = jax.lax.broadcasted_iota(jnp.int32, (1, LB), 1)
        wpos = lane % W
        hpos = (lane // W) % H
        left = jnp.where(
            wpos >= 1,
            jnp.concatenate([xb[:, LB - 1:], xb[:, :LB - 1]], axis=1),
            jnp.bfloat16(0))
        right = jnp.where(
            wpos <= W - 2,
            jnp.concatenate([xb[:, 1:], xb[:, :1]], axis=1),
            jnp.bfloat16(0))
        a3 = jnp.concatenate([left, xb, right], axis=0)
        c = jax.lax.dot_general(wc_ref[...], a3, (((1,), (0,)), ((), ())),
                                preferred_element_type=jnp.float32)
        c0 = c[:Cout]
        c2 = c[2 * Cout:]
        up = jnp.where(
            hpos >= 1,
            jnp.concatenate([c0[:, LB - W:], c0[:, :LB - W]], axis=1), 0.0)
        dn = jnp.where(
            hpos <= H - 2,
            jnp.concatenate([c2[:, W:], c2[:, :W]], axis=1), 0.0)
        y = c[Cout:2 * Cout] + up + dn
        s = jnp.sum(y, axis=1, keepdims=True)
        ss = jnp.sum(y * y, axis=1, keepdims=True)
        acc1_ref[...] += jnp.concatenate([s, ss], axis=1)
        y1_ref[i] = y.astype(jnp.bfloat16)

    @pl.when(jnp.logical_and(ph == 1, i == 0))
    def _fold1():
        tot = acc1_ref[...]
        mean = tot[:, 0:1] / count
        var = tot[:, 1:2] / count - mean * mean
        sc = g1_ref[...] * jax.lax.rsqrt(var + _EPS)
        sc1_ref[...] = sc
        sh1_ref[...] = b1_ref[...] - mean * sc

    @pl.when(ph == 1)
    def _stats2():
        a = jnp.maximum(
            y1_ref[i].astype(jnp.float32) * sc1_ref[...] + sh1_ref[...], 0.0)
        z = jax.lax.dot_general(w1t_ref[...], a.astype(jnp.bfloat16),
                                (((1,), (0,)), ((), ())),
                                preferred_element_type=jnp.float32)
        s = jnp.sum(z, axis=1, keepdims=True)
        ss = jnp.sum(z * z, axis=1, keepdims=True)
        acc2_ref[...] += jnp.concatenate([s, ss], axis=1)
        y1_ref[i] = z.astype(jnp.bfloat16)

    @pl.when(jnp.logical_and(ph == 2, i == 0))
    def _fold2():
        tot = acc2_ref[...]
        mean = tot[:, 0:1] / count
        var = tot[:, 1:2] / count - mean * mean
        sc = g2_ref[...] * jax.lax.rsqrt(var + _EPS)
        sc2_ref[...] = sc
        sh2_ref[...] = b2_ref[...] - mean * sc

    @pl.when(ph == 2)
    def _emit():
        z = y1_ref[i].astype(jnp.float32)
        o = jnp.maximum(z * sc2_ref[...] + sh2_ref[...],
                        0.0).astype(jnp.bfloat16)
        for b in range(B):
            o_ref[b] = o[:, b * HW:(b + 1) * HW]


@jax.jit
def _forward(x_nchw, w3_hwio, w1, gamma1, beta1, gamma2, beta2):
    N, Cin, H, W = x_nchw.shape
    Cout = w3_hwio.shape[-1]
    HW = H * W
    B = 64 if N % 64 == 0 else (8 if N % 8 == 0 else 1)
    S = N // B
    LB = B * HW

    x3 = x_nchw.reshape(N, Cin, HW)
    wc = jnp.transpose(w3_hwio, (0, 3, 1, 2)).reshape(
        3 * Cout, 3 * Cin).astype(jnp.bfloat16)
    w1t = jnp.transpose(w1).astype(jnp.bfloat16)

    out3 = pl.pallas_call(
        functools.partial(_block_kernel, B=B, H=H, W=W, Cout=Cout,
                          count=N * HW),
        grid=(3, S),
        in_specs=[
            pl.BlockSpec((B, Cin, HW),
                         lambda p, i: (jnp.where(p == 0, i, 0), 0, 0)),
            pl.BlockSpec((3 * Cout, 3 * Cin), lambda p, i: (0, 0)),
            pl.BlockSpec((Cout, Cout), lambda p, i: (0, 0)),
            pl.BlockSpec((Cout, 1), lambda p, i: (0, 0)),
            pl.BlockSpec((Cout, 1), lambda p, i: (0, 0)),
            pl.BlockSpec((Cout, 1), lambda p, i: (0, 0)),
            pl.BlockSpec((Cout, 1), lambda p, i: (0, 0)),
        ],
        out_specs=pl.BlockSpec((B, Cout, HW),
                               lambda p, i: (jnp.where(p == 2, i, 0), 0, 0)),
        out_shape=jax.ShapeDtypeStruct((N, Cout, HW), jnp.bfloat16),
        scratch_shapes=[
            pltpu.VMEM((S, Cout, LB), jnp.bfloat16),
            pltpu.VMEM((Cout, 2), jnp.float32),
            pltpu.VMEM((Cout, 2), jnp.float32),
            pltpu.VMEM((Cout, 1), jnp.float32),
            pltpu.VMEM((Cout, 1), jnp.float32),
            pltpu.VMEM((Cout, 1), jnp.float32),
            pltpu.VMEM((Cout, 1), jnp.float32),
        ],
        compiler_params=pltpu.CompilerParams(
            dimension_semantics=("arbitrary", "arbitrary")),
    )(x3, wc, w1t, gamma1.reshape(-1, 1), beta1.reshape(-1, 1),
      gamma2.reshape(-1, 1), beta2.reshape(-1, 1))

    return out3.reshape(N, Cout, H, W).astype(jnp.float32)


def kernel(x_nchw, w3_hwio, w1, gamma1, beta1, gamma2, beta2):
    return _forward(x_nchw, w3_hwio, w1, gamma1, beta1, gamma2, beta2)
